# baseline (device time: 26513 ns/iter reference)
import jax
import jax.numpy as jnp
from jax import lax
from jax.experimental import pallas as pl
from jax.experimental.pallas import tpu as pltpu

BM = 512


def _body(x_ref, dy_ref, out_ref, acc_ref, comm_ref, send_sem, recv_sem):
    i = pl.program_id(0)
    n_steps = pl.num_programs(0)

    x = x_ref[...]
    dy = dy_ref[...]
    d = x.shape[1]
    mu = jnp.sum(x, axis=1, keepdims=True) * (1.0 / d)
    xc = x - mu
    var = jnp.sum(xc * xc, axis=1, keepdims=True) * (1.0 / d)
    rstd = lax.rsqrt(var + 1e-5)
    xhat = xc * rstd
    dgamma = jnp.sum(dy * xhat, axis=0, keepdims=True)
    dbeta = jnp.sum(dy, axis=0, keepdims=True)
    part = jnp.concatenate([dgamma, dbeta], axis=0)

    @pl.when(i == 0)
    def _():
        acc_ref[...] = part

    @pl.when(i != 0)
    def _():
        acc_ref[...] += part

    @pl.when(i == n_steps - 1)
    def _():
        my_x = lax.axis_index("x")
        my_y = lax.axis_index("y")
        peer = (1 - my_x, my_y)

        barrier = pltpu.get_barrier_semaphore()
        pl.semaphore_signal(
            barrier, inc=1, device_id=peer, device_id_type=pl.DeviceIdType.MESH
        )
        pl.semaphore_wait(barrier, 1)

        rdma = pltpu.make_async_remote_copy(
            src_ref=acc_ref,
            dst_ref=comm_ref,
            send_sem=send_sem,
            recv_sem=recv_sem,
            device_id=peer,
            device_id_type=pl.DeviceIdType.MESH,
        )
        rdma.start()
        rdma.wait()

        out_ref[...] = acc_ref[...] + comm_ref[...]


def kernel(x, dy, gamma):
    del gamma
    m, d = x.shape

    return pl.pallas_call(
        _body,
        grid=(m // BM,),
        in_specs=[
            pl.BlockSpec((BM, d), lambda i: (i, 0)),
            pl.BlockSpec((BM, d), lambda i: (i, 0)),
        ],
        out_specs=pl.BlockSpec((2, d), lambda i: (0, 0)),
        out_shape=jax.ShapeDtypeStruct((2, d), jnp.float32),
        scratch_shapes=[
            pltpu.VMEM((2, d), jnp.float32),
            pltpu.VMEM((2, d), jnp.float32),
            pltpu.SemaphoreType.DMA,
            pltpu.SemaphoreType.DMA,
        ],
        compiler_params=pltpu.CompilerParams(collective_id=0),
    )(x, dy)


# device time: 18866 ns/iter; 1.4053x vs baseline; 1.4053x over previous
import jax
import jax.numpy as jnp
from jax import lax
from jax.experimental import pallas as pl
from jax.experimental.pallas import tpu as pltpu

BM = 512


def _body(y_ref, x_ref, dy_ref, out_ref, acc_ref, comm_ref, send_sems, recv_sems):
    i = pl.program_id(0)
    n_steps = pl.num_programs(0)

    x = x_ref[...]
    dy = dy_ref[...]
    d = x.shape[1]
    mu = jnp.sum(x, axis=1, keepdims=True) * (1.0 / d)
    xc = x - mu
    var = jnp.sum(xc * xc, axis=1, keepdims=True) * (1.0 / d)
    rstd = lax.rsqrt(var + 1e-5)
    xhat = xc * rstd
    dgamma = jnp.sum(dy * xhat, axis=0, keepdims=True)
    dbeta = jnp.sum(dy, axis=0, keepdims=True)
    part = jnp.concatenate([dgamma, dbeta], axis=0)

    @pl.when(i == 0)
    def _():
        acc_ref[...] = part

    @pl.when(i != 0)
    def _():
        acc_ref[...] += part

    @pl.when(i == n_steps - 1)
    def _():
        my_x = lax.axis_index("x")
        my_y = lax.axis_index("y")
        x_peer = (1 - my_x, my_y)
        y_peer = (my_x, 1 - my_y)

        barrier = pltpu.get_barrier_semaphore()
        for peer in (x_peer, y_peer):
            pl.semaphore_signal(
                barrier, inc=1, device_id=peer,
                device_id_type=pl.DeviceIdType.MESH,
            )
        pl.semaphore_wait(barrier, 2)

        rdma0 = pltpu.make_async_remote_copy(
            src_ref=acc_ref,
            dst_ref=comm_ref.at[0],
            send_sem=send_sems.at[0],
            recv_sem=recv_sems.at[0],
            device_id=x_peer,
            device_id_type=pl.DeviceIdType.MESH,
        )
        rdma0.start()
        rdma0.wait()
        acc_ref[...] += comm_ref[0]

        rdma1 = pltpu.make_async_remote_copy(
            src_ref=acc_ref,
            dst_ref=comm_ref.at[1],
            send_sem=send_sems.at[1],
            recv_sem=recv_sems.at[1],
            device_id=y_peer,
            device_id_type=pl.DeviceIdType.MESH,
        )
        rdma1.start()
        rdma1.wait()

        out_ref[...] = acc_ref[...] + comm_ref[1]


def kernel(x, dy, gamma):
    del gamma
    m, d = x.shape
    half = m // 2
    n_blk = half // BM

    y_idx = lax.axis_index("y").astype(jnp.int32).reshape((1,))

    grid_spec = pltpu.PrefetchScalarGridSpec(
        num_scalar_prefetch=1,
        grid=(n_blk,),
        in_specs=[
            pl.BlockSpec((BM, d), lambda i, y_ref: (y_ref[0] * n_blk + i, 0)),
            pl.BlockSpec((BM, d), lambda i, y_ref: (y_ref[0] * n_blk + i, 0)),
        ],
        out_specs=pl.BlockSpec((2, d), lambda i, y_ref: (0, 0)),
        scratch_shapes=[
            pltpu.VMEM((2, d), jnp.float32),
            pltpu.VMEM((2, 2, d), jnp.float32),
            pltpu.SemaphoreType.DMA((2,)),
            pltpu.SemaphoreType.DMA((2,)),
        ],
    )

    return pl.pallas_call(
        _body,
        grid_spec=grid_spec,
        out_shape=jax.ShapeDtypeStruct((2, d), jnp.float32),
        compiler_params=pltpu.CompilerParams(collective_id=0),
    )(y_idx, x, dy)


# device time: 18093 ns/iter; 1.4654x vs baseline; 1.0427x over previous
import jax
import jax.numpy as jnp
from jax import lax
from jax.experimental import pallas as pl
from jax.experimental.pallas import tpu as pltpu

BM = 512


def _body(y_ref, x_ref, dy_ref, out_ref, acc_ref, comm_ref, send_sems, recv_sems):
    i = pl.program_id(0)
    n_steps = pl.num_programs(0)

    x = x_ref[...]
    dy = dy_ref[...]
    d = x.shape[1]
    mu = jnp.sum(x, axis=1, keepdims=True) * (1.0 / d)
    xc = x - mu
    var = jnp.sum(xc * xc, axis=1, keepdims=True) * (1.0 / d)
    rstd = lax.rsqrt(var + 1e-5)
    xhat = xc * rstd
    dgamma = jnp.sum(dy * xhat, axis=0, keepdims=True)
    dbeta = jnp.sum(dy, axis=0, keepdims=True)
    part = jnp.concatenate([dgamma, dbeta], axis=0)

    @pl.when(i == 0)
    def _():
        acc_ref[...] = part

    @pl.when(i != 0)
    def _():
        acc_ref[...] += part

    @pl.when(i == n_steps - 1)
    def _():
        my_x = lax.axis_index("x")
        my_y = lax.axis_index("y")
        peers = (
            (1 - my_x, my_y),
            (my_x, 1 - my_y),
            (1 - my_x, 1 - my_y),
        )

        barrier = pltpu.get_barrier_semaphore()
        for peer in peers:
            pl.semaphore_signal(
                barrier, inc=1, device_id=peer,
                device_id_type=pl.DeviceIdType.MESH,
            )
        pl.semaphore_wait(barrier, 3)

        rdmas = []
        for k, peer in enumerate(peers):
            rdma = pltpu.make_async_remote_copy(
                src_ref=acc_ref,
                dst_ref=comm_ref.at[k],
                send_sem=send_sems.at[k],
                recv_sem=recv_sems.at[k],
                device_id=peer,
                device_id_type=pl.DeviceIdType.MESH,
            )
            rdma.start()
            rdmas.append(rdma)
        for rdma in rdmas:
            rdma.wait()

        out_ref[...] = (
            (acc_ref[...] + comm_ref[0]) + (comm_ref[1] + comm_ref[2])
        )


def kernel(x, dy, gamma):
    del gamma
    m, d = x.shape
    half = m // 2
    n_blk = half // BM

    y_idx = lax.axis_index("y").astype(jnp.int32).reshape((1,))

    grid_spec = pltpu.PrefetchScalarGridSpec(
        num_scalar_prefetch=1,
        grid=(n_blk,),
        in_specs=[
            pl.BlockSpec((BM, d), lambda i, y_ref: (y_ref[0] * n_blk + i, 0)),
            pl.BlockSpec((BM, d), lambda i, y_ref: (y_ref[0] * n_blk + i, 0)),
        ],
        out_specs=pl.BlockSpec((2, d), lambda i, y_ref: (0, 0)),
        scratch_shapes=[
            pltpu.VMEM((2, d), jnp.float32),
            pltpu.VMEM((3, 2, d), jnp.float32),
            pltpu.SemaphoreType.DMA((3,)),
            pltpu.SemaphoreType.DMA((3,)),
        ],
    )

    return pl.pallas_call(
        _body,
        grid_spec=grid_spec,
        out_shape=jax.ShapeDtypeStruct((2, d), jnp.float32),
        compiler_params=pltpu.CompilerParams(collective_id=0),
    )(y_idx, x, dy)
